# baseline (speedup 1.0000x reference)
import jax
import jax.numpy as jnp
from jax import lax
from jax.experimental import pallas as pl
from jax.experimental.pallas import tpu as pltpu
from jax.experimental.pallas import tpu_sc as plsc

VAR_LEN = 1000000
EMBED_SIZE = 64
BATCH = 16384
FIELDS = 26

NUM_CORES = 2
NUM_SUBCORES = 16
NUM_WORKERS = NUM_CORES * NUM_SUBCORES

B_TOTAL = BATCH * FIELDS
B_PER_W = B_TOTAL // NUM_WORKERS
BATCH_PER_W = BATCH // NUM_WORKERS
B_LOC = 256
GPITCH = 68

CHUNK = 208
N_CHUNKS = B_PER_W // CHUNK
NBUF = 8
LOOKAHEAD = 4
N_BLOCKS = N_CHUNKS // NBUF


def _gather_body(idx_hbm, table_hbm, out_hbm, idx_v, rows_v, gsems, wsems):
    wid = lax.axis_index("s") * NUM_CORES + lax.axis_index("c")
    base = pl.multiple_of(wid * B_PER_W, B_PER_W)
    pltpu.sync_copy(idx_hbm.at[pl.ds(base, B_PER_W)], idx_v)

    def gather_start(j, b):
        off = pl.multiple_of(j * CHUNK, CHUNK)
        pltpu.make_async_copy(
            table_hbm.at[idx_v.at[pl.ds(off, CHUNK)]], rows_v.at[b], gsems.at[b]
        ).start()

    def gather_wait(b):
        pltpu.make_async_copy(
            table_hbm.at[idx_v.at[pl.ds(0, CHUNK)]], rows_v.at[b], gsems.at[b]
        ).wait()

    def write_start(j, b):
        off = pl.multiple_of(j * CHUNK, CHUNK)
        pltpu.make_async_copy(
            rows_v.at[b], out_hbm.at[pl.ds(base + off, CHUNK)], wsems.at[b]
        ).start()

    def write_wait(b):
        pltpu.make_async_copy(
            rows_v.at[b], out_hbm.at[pl.ds(base, CHUNK)], wsems.at[b]
        ).wait()

    for b in range(LOOKAHEAD):
        gather_start(b, b)

    def run_chunk(j, b, fire, drain):
        fb = (b + LOOKAHEAD) % NBUF
        if fire:
            if drain:
                write_wait(fb)
            gather_start(j + LOOKAHEAD, fb)
        gather_wait(b)
        write_start(j, b)

    for b in range(NBUF):
        run_chunk(b, b, fire=True, drain=(b + LOOKAHEAD >= NBUF))

    def block(gi, carry):
        g = gi * NBUF
        for b in range(NBUF):
            run_chunk(g + b, b, fire=True, drain=True)
        return carry

    lax.fori_loop(1, N_BLOCKS - 1, block, 0)

    g = N_CHUNKS - NBUF
    for b in range(NBUF):
        run_chunk(g + b, b, fire=(b + LOOKAHEAD < NBUF), drain=(b + LOOKAHEAD < NBUF))

    for b in range(NBUF):
        write_wait(b)


def _format_body(rows_hbm, out_hbm, gbuf, tbuf, rsems, wsems):
    wid = lax.axis_index("s") * NUM_CORES + lax.axis_index("c")
    iota16 = lax.iota(jnp.int32, 16)

    def read_start(f, half, pg):
        pltpu.make_async_copy(
            rows_hbm.at[wid, pl.ds(B_LOC * half, B_LOC), f, :],
            gbuf.at[pg, :, pl.ds(0, EMBED_SIZE)],
            rsems.at[pg],
        ).start()

    def read_wait(pg):
        pltpu.make_async_copy(
            rows_hbm.at[0, pl.ds(0, B_LOC), 0, :],
            gbuf.at[pg, :, pl.ds(0, EMBED_SIZE)],
            rsems.at[pg],
        ).wait()

    def transpose(pg, pt):
        @plsc.parallel_loop(0, 128, unroll=2)
        def _tr(q):
            r = q >> 4
            cc = (q >> 3) & 1
            s = q & 7
            cs = 8 * r + s + jnp.zeros((16,), jnp.int32)
            rbase = iota16 + 128 * cc
            for g in range(8):
                v = plsc.load_gather(gbuf.at[pg], [rbase + 16 * g, cs])
                tbuf[pt, r, cc, s, pl.ds(16 * g, 16)] = v

    def write_start(f, half, pt):
        c0 = 4 * wid + 2 * half
        pltpu.make_async_copy(
            tbuf.at[pt], out_hbm.at[f, :, pl.ds(c0, 2)], wsems.at[pt]
        ).start()

    def write_wait(pt):
        pltpu.make_async_copy(
            tbuf.at[pt], out_hbm.at[0, :, pl.ds(0, 2)], wsems.at[pt]
        ).wait()

    def step(f, half, nf, nhalf, k, fire, drain):
        pg, pt = k % 4, k % 2
        if fire:
            read_start(nf, nhalf, (k + 3) % 4)
        read_wait(pg)
        if drain:
            write_wait(pt)
        transpose(pg, pt)
        write_start(f, half, pt)

    read_start(0, 0, 0)
    read_start(0, 1, 1)
    read_start(1, 0, 2)
    step(0, 0, 1, 1, 0, fire=True, drain=False)
    step(0, 1, 2, 0, 1, fire=True, drain=False)
    step(1, 0, 2, 1, 2, fire=True, drain=True)
    step(1, 1, 3, 0, 3, fire=True, drain=True)

    def quad(q, carry):
        f = 2 * q
        step(f, 0, f + 1, 1, 0, fire=True, drain=True)
        step(f, 1, f + 2, 0, 1, fire=True, drain=True)
        step(f + 1, 0, f + 2, 1, 2, fire=True, drain=True)
        step(f + 1, 1, f + 3, 0, 3, fire=True, drain=True)
        return carry

    lax.fori_loop(1, 12, quad, 0)

    step(24, 0, 25, 1, 0, fire=True, drain=True)
    step(24, 1, 0, 0, 1, fire=False, drain=True)
    step(25, 0, 0, 0, 2, fire=False, drain=True)
    step(25, 1, 0, 0, 3, fire=False, drain=True)

    write_wait(0)
    write_wait(1)


RT_LANES = 16384
RT_GRID = 62


def _retile_body(wt_ref, out_ref):
    eye = jnp.eye(EMBED_SIZE, dtype=jnp.float32)
    xt = jax.lax.dot_general(
        wt_ref[...], eye, (((0,), (0,)), ((), ())),
        preferred_element_type=jnp.float32,
    ).reshape(RT_LANES // 2, 2, EMBED_SIZE)
    out_ref[:, :EMBED_SIZE] = xt[:, 0, :]
    out_ref[:, EMBED_SIZE:] = xt[:, 1, :]


def _retile(weight):
    wt = weight.T
    return pl.pallas_call(
        _retile_body,
        grid=(RT_GRID,),
        in_specs=[pl.BlockSpec((EMBED_SIZE, RT_LANES), lambda g: (0, g))],
        out_specs=pl.BlockSpec((RT_LANES // 2, 2 * EMBED_SIZE), lambda g: (g, 0)),
        out_shape=jax.ShapeDtypeStruct((VAR_LEN // 2, 2 * EMBED_SIZE), jnp.float32),
    )(wt)


@jax.jit
def _emb(idx_perm, weight):
    weight = _retile(weight).reshape(VAR_LEN, EMBED_SIZE)
    mesh = plsc.VectorSubcoreMesh(core_axis_name="c", subcore_axis_name="s")
    rows = pl.kernel(
        _gather_body,
        out_type=jax.ShapeDtypeStruct((B_TOTAL, EMBED_SIZE), jnp.float32),
        mesh=mesh,
        scratch_types=[
            pltpu.VMEM((B_PER_W,), jnp.int32),
            pltpu.VMEM((NBUF, CHUNK, EMBED_SIZE), jnp.float32),
            pltpu.SemaphoreType.DMA((NBUF,)),
            pltpu.SemaphoreType.DMA((NBUF,)),
        ],
        compiler_params=pltpu.CompilerParams(use_tc_tiling_on_sc=False),
    )(idx_perm, weight)

    mesh2 = plsc.VectorSubcoreMesh(core_axis_name="c", subcore_axis_name="s")
    return pl.kernel(
        _format_body,
        out_type=jax.ShapeDtypeStruct((FIELDS, 8, 128, 8, 128), jnp.float32),
        mesh=mesh2,
        scratch_types=[
            pltpu.VMEM((4, B_LOC, GPITCH), jnp.float32),
            pltpu.VMEM((2, 8, 2, 8, 128), jnp.float32),
            pltpu.SemaphoreType.DMA((4,)),
            pltpu.SemaphoreType.DMA((2,)),
        ],
        compiler_params=pltpu.CompilerParams(
            use_tc_tiling_on_sc=False, needs_layout_passes=False
        ),
    )(rows.reshape(NUM_WORKERS, BATCH_PER_W, FIELDS, EMBED_SIZE))


def kernel(input, weight):
    out5 = _emb(input.reshape(-1).astype(jnp.int32), weight)
    return out5.transpose(2, 4, 0, 1, 3).reshape(BATCH, FIELDS, EMBED_SIZE)


# speedup vs baseline: 1.0790x; 1.0790x over previous
import jax
import jax.numpy as jnp
from jax import lax
from jax.experimental import pallas as pl
from jax.experimental.pallas import tpu as pltpu
from jax.experimental.pallas import tpu_sc as plsc

VAR_LEN = 1000000
EMBED_SIZE = 64
BATCH = 16384
FIELDS = 26

NUM_CORES = 2
NUM_SUBCORES = 16
NUM_WORKERS = NUM_CORES * NUM_SUBCORES

B_TOTAL = BATCH * FIELDS
B_PER_W = B_TOTAL // NUM_WORKERS
BATCH_PER_W = BATCH // NUM_WORKERS
B_LOC = 256
GPITCH = 68

CHUNK = 208
N_CHUNKS = B_PER_W // CHUNK
NBUF = 8
LOOKAHEAD = 4
N_BLOCKS = N_CHUNKS // NBUF


def _gather_body(idx_hbm, table_hbm, out_hbm, idx_v, rows_v, gsems, wsems):
    wid = lax.axis_index("s") * NUM_CORES + lax.axis_index("c")
    base = pl.multiple_of(wid * B_PER_W, B_PER_W)
    pltpu.sync_copy(idx_hbm.at[pl.ds(base, B_PER_W)], idx_v)

    def gather_start(j, b):
        off = pl.multiple_of(j * CHUNK, CHUNK)
        pltpu.make_async_copy(
            table_hbm.at[idx_v.at[pl.ds(off, CHUNK)]], rows_v.at[b], gsems.at[b]
        ).start()

    def gather_wait(b):
        pltpu.make_async_copy(
            table_hbm.at[idx_v.at[pl.ds(0, CHUNK)]], rows_v.at[b], gsems.at[b]
        ).wait()

    def write_start(j, b):
        off = pl.multiple_of(j * CHUNK, CHUNK)
        pltpu.make_async_copy(
            rows_v.at[b], out_hbm.at[pl.ds(base + off, CHUNK)], wsems.at[b]
        ).start()

    def write_wait(b):
        pltpu.make_async_copy(
            rows_v.at[b], out_hbm.at[pl.ds(base, CHUNK)], wsems.at[b]
        ).wait()

    for b in range(LOOKAHEAD):
        gather_start(b, b)

    def run_chunk(j, b, fire, drain):
        fb = (b + LOOKAHEAD) % NBUF
        if fire:
            if drain:
                write_wait(fb)
            gather_start(j + LOOKAHEAD, fb)
        gather_wait(b)
        write_start(j, b)

    for b in range(NBUF):
        run_chunk(b, b, fire=True, drain=(b + LOOKAHEAD >= NBUF))

    def block(gi, carry):
        g = gi * NBUF
        for b in range(NBUF):
            run_chunk(g + b, b, fire=True, drain=True)
        return carry

    lax.fori_loop(1, N_BLOCKS - 1, block, 0)

    g = N_CHUNKS - NBUF
    for b in range(NBUF):
        run_chunk(g + b, b, fire=(b + LOOKAHEAD < NBUF), drain=(b + LOOKAHEAD < NBUF))

    for b in range(NBUF):
        write_wait(b)


def _format_body(rows_hbm, out_hbm, gbuf, tbuf, rsems, wsems):
    wid = lax.axis_index("s") * NUM_CORES + lax.axis_index("c")
    iota16 = lax.iota(jnp.int32, 16)

    def read_start(f, half, pg):
        pltpu.make_async_copy(
            rows_hbm.at[wid, pl.ds(B_LOC * half, B_LOC), f, :],
            gbuf.at[pg, :, pl.ds(0, EMBED_SIZE)],
            rsems.at[pg],
        ).start()

    def read_wait(pg):
        pltpu.make_async_copy(
            rows_hbm.at[0, pl.ds(0, B_LOC), 0, :],
            gbuf.at[pg, :, pl.ds(0, EMBED_SIZE)],
            rsems.at[pg],
        ).wait()

    def transpose(pg, pt):
        @plsc.parallel_loop(0, 128, unroll=2)
        def _tr(q):
            r = q >> 4
            cc = (q >> 3) & 1
            s = q & 7
            cs = 8 * r + s + jnp.zeros((16,), jnp.int32)
            rbase = iota16 + 128 * cc
            for g in range(8):
                v = plsc.load_gather(gbuf.at[pg], [rbase + 16 * g, cs])
                tbuf[pt, r, cc, s, pl.ds(16 * g, 16)] = v

    def write_start(f, half, pt):
        c0 = 4 * wid + 2 * half
        pltpu.make_async_copy(
            tbuf.at[pt], out_hbm.at[f, :, pl.ds(c0, 2)], wsems.at[pt]
        ).start()

    def write_wait(pt):
        pltpu.make_async_copy(
            tbuf.at[pt], out_hbm.at[0, :, pl.ds(0, 2)], wsems.at[pt]
        ).wait()

    def step(f, half, nf, nhalf, k, fire, drain):
        pg, pt = k % 4, k % 2
        if fire:
            read_start(nf, nhalf, (k + 3) % 4)
        read_wait(pg)
        if drain:
            write_wait(pt)
        transpose(pg, pt)
        write_start(f, half, pt)

    read_start(0, 0, 0)
    read_start(0, 1, 1)
    read_start(1, 0, 2)
    step(0, 0, 1, 1, 0, fire=True, drain=False)
    step(0, 1, 2, 0, 1, fire=True, drain=False)
    step(1, 0, 2, 1, 2, fire=True, drain=True)
    step(1, 1, 3, 0, 3, fire=True, drain=True)

    def quad(q, carry):
        f = 2 * q
        step(f, 0, f + 1, 1, 0, fire=True, drain=True)
        step(f, 1, f + 2, 0, 1, fire=True, drain=True)
        step(f + 1, 0, f + 2, 1, 2, fire=True, drain=True)
        step(f + 1, 1, f + 3, 0, 3, fire=True, drain=True)
        return carry

    lax.fori_loop(1, 12, quad, 0)

    step(24, 0, 25, 1, 0, fire=True, drain=True)
    step(24, 1, 0, 0, 1, fire=False, drain=True)
    step(25, 0, 0, 0, 2, fire=False, drain=True)
    step(25, 1, 0, 0, 3, fire=False, drain=True)

    write_wait(0)
    write_wait(1)


RT_LANES = 16384
RT_GRID = 62


def _retile_body(wt_ref, out_ref):
    xt = wt_ref[...].T.reshape(RT_LANES // 2, 2, EMBED_SIZE)
    out_ref[:, :EMBED_SIZE] = xt[:, 0, :]
    out_ref[:, EMBED_SIZE:] = xt[:, 1, :]


def _retile(weight):
    wt = weight.T
    return pl.pallas_call(
        _retile_body,
        grid=(RT_GRID,),
        in_specs=[pl.BlockSpec((EMBED_SIZE, RT_LANES), lambda g: (0, g))],
        out_specs=pl.BlockSpec((RT_LANES // 2, 2 * EMBED_SIZE), lambda g: (g, 0)),
        out_shape=jax.ShapeDtypeStruct((VAR_LEN // 2, 2 * EMBED_SIZE), jnp.float32),
    )(wt)


@jax.jit
def _emb(idx_perm, weight):
    weight = _retile(weight).reshape(VAR_LEN, EMBED_SIZE)
    mesh = plsc.VectorSubcoreMesh(core_axis_name="c", subcore_axis_name="s")
    rows = pl.kernel(
        _gather_body,
        out_type=jax.ShapeDtypeStruct((B_TOTAL, EMBED_SIZE), jnp.float32),
        mesh=mesh,
        scratch_types=[
            pltpu.VMEM((B_PER_W,), jnp.int32),
            pltpu.VMEM((NBUF, CHUNK, EMBED_SIZE), jnp.float32),
            pltpu.SemaphoreType.DMA((NBUF,)),
            pltpu.SemaphoreType.DMA((NBUF,)),
        ],
        compiler_params=pltpu.CompilerParams(use_tc_tiling_on_sc=False),
    )(idx_perm, weight)

    mesh2 = plsc.VectorSubcoreMesh(core_axis_name="c", subcore_axis_name="s")
    return pl.kernel(
        _format_body,
        out_type=jax.ShapeDtypeStruct((FIELDS, 8, 128, 8, 128), jnp.float32),
        mesh=mesh2,
        scratch_types=[
            pltpu.VMEM((4, B_LOC, GPITCH), jnp.float32),
            pltpu.VMEM((2, 8, 2, 8, 128), jnp.float32),
            pltpu.SemaphoreType.DMA((4,)),
            pltpu.SemaphoreType.DMA((2,)),
        ],
        compiler_params=pltpu.CompilerParams(
            use_tc_tiling_on_sc=False, needs_layout_passes=False
        ),
    )(rows.reshape(NUM_WORKERS, BATCH_PER_W, FIELDS, EMBED_SIZE))


def kernel(input, weight):
    out5 = _emb(input.reshape(-1).astype(jnp.int32), weight)
    return out5.transpose(2, 4, 0, 1, 3).reshape(BATCH, FIELDS, EMBED_SIZE)
